# baseline (speedup 1.0000x reference)
import functools

import jax
import jax.numpy as jnp
from jax import lax
from jax.experimental import pallas as pl
from jax.experimental.pallas import tpu as pltpu
from jax.experimental.pallas import tpu_sc as plsc

N0 = 10000
E0 = 160000
E1 = 1280000
D = 128

N0P = 10240
R0 = E0 // 128
R1 = E1 // 128
NSLICE = 16
SW = D // NSLICE

_MESH = plsc.VectorSubcoreMesh(core_axis_name="c", subcore_axis_name="s")


def _fill_ones(ref):
    for k in range(8):
        ref[0, pl.ds(16 * k, 16)] = jnp.ones((16,), jnp.float32)


def _zero_fill_1d(ref, n):
    z = jnp.zeros((16,), jnp.float32)

    def b(i, carry):
        ref[pl.ds(16 * i, 16)] = z
        return carry

    lax.fori_loop(0, n // 16, b, 0)


def _zero_fill_2d(ref, rows, width):
    z = jnp.zeros((16,), jnp.float32)

    def b(r, carry):
        for k in range(width // 16):
            ref[r, pl.ds(16 * k, 16)] = z
        return carry

    lax.fori_loop(0, rows, b, 0)


def _sc_pre_body(h0, h1, s1r, d1r, s0r64, d0r64,
                 gath, g0p, dout0p, din0p, dout1p, din1p,
                 sidx, didx, sidx64, didx64, h1rows, h0rows, ones, ones64,
                 zbuf1d,
                 dout0_sh, din0_sh, dout1_sh, din1_sh, g0_sh):
    c = lax.axis_index("c")
    s = lax.axis_index("s")
    _fill_ones(ones)
    for k in range(4):
        ones64[0, pl.ds(16 * k, 16)] = jnp.ones((16,), jnp.float32)
    _zero_fill_1d(zbuf1d, 2000)
    _zero_fill_2d(h1rows, 64, D)
    pltpu.sync_copy(zbuf1d.at[pl.ds(0, 640)], dout0_sh.at[pl.ds(s * 640, 640)])
    pltpu.sync_copy(zbuf1d.at[pl.ds(0, 640)], din0_sh.at[pl.ds(s * 640, 640)])
    for j in range(5):
        pltpu.sync_copy(zbuf1d, dout1_sh.at[pl.ds(s * 10000 + 2000 * j, 2000)])
        pltpu.sync_copy(zbuf1d, din1_sh.at[pl.ds(s * 10000 + 2000 * j, 2000)])
    for j in range(10):
        pltpu.sync_copy(h1rows, g0_sh.at[pl.ds(s * 640 + 64 * j, 64)])
    plsc.subcore_barrier()

    def e1_body(t, carry):
        r = s + 16 * t

        @pl.when(r < 5000)
        def _():
            row = c * 5000 + r
            pltpu.sync_copy(s1r.at[pl.ds(row, 1)], sidx)
            pltpu.sync_copy(d1r.at[pl.ds(row, 1)], didx)
            pltpu.sync_copy(ones.at[0], dout1_sh.at[sidx.at[0]], add=True)
            pltpu.sync_copy(ones.at[0], din1_sh.at[didx.at[0]], add=True)

        return carry

    lax.fori_loop(0, 313, e1_body, 0)

    def e0_body(t, carry):
        r = s + 16 * t

        @pl.when(r < 1250)
        def _():
            row = c * 1250 + r
            pltpu.sync_copy(s0r64.at[pl.ds(row, 1)], sidx64)
            pltpu.sync_copy(d0r64.at[pl.ds(row, 1)], didx64)
            pltpu.sync_copy(ones64.at[0], dout0_sh.at[sidx64.at[0]], add=True)
            pltpu.sync_copy(ones64.at[0], din0_sh.at[didx64.at[0]], add=True)
            pltpu.sync_copy(h1.at[pl.ds(row * 64, 64)], h1rows)
            pltpu.sync_copy(h1rows, g0_sh.at[didx64.at[0]], add=True)
            pltpu.sync_copy(h0.at[didx64.at[0]], h0rows)
            pltpu.sync_copy(h0rows, gath.at[pl.ds(row * 64, 64)])

        return carry

    lax.fori_loop(0, 79, e0_body, 0)
    plsc.subcore_barrier()

    def _out1d_640(sh, out):
        pltpu.sync_copy(sh.at[pl.ds(s * 640, 640)], zbuf1d.at[pl.ds(0, 640)])
        pltpu.sync_copy(zbuf1d.at[pl.ds(0, 640)],
                        out.at[pl.ds(c * N0P + s * 640, 640)])

    def _out1d_10000(sh, out):
        for j in range(5):
            pltpu.sync_copy(sh.at[pl.ds(s * 10000 + 2000 * j, 2000)], zbuf1d)
            pltpu.sync_copy(
                zbuf1d, out.at[pl.ds(c * E0 + s * 10000 + 2000 * j, 2000)])

    _out1d_640(dout0_sh, dout0p)
    _out1d_640(din0_sh, din0p)
    _out1d_10000(dout1_sh, dout1p)
    _out1d_10000(din1_sh, din1p)
    for j in range(10):
        pltpu.sync_copy(g0_sh.at[pl.ds(s * 640 + 64 * j, 64)], h1rows)
        pltpu.sync_copy(h1rows,
                        g0p.at[pl.ds(c * N0P + s * 640 + 64 * j, 64)])


_sc_pre = pl.kernel(
    _sc_pre_body,
    out_type=[
        jax.ShapeDtypeStruct((E0, D), jnp.float32),
        jax.ShapeDtypeStruct((2 * N0P, D), jnp.float32),
        jax.ShapeDtypeStruct((2 * N0P,), jnp.float32),
        jax.ShapeDtypeStruct((2 * N0P,), jnp.float32),
        jax.ShapeDtypeStruct((2 * E0,), jnp.float32),
        jax.ShapeDtypeStruct((2 * E0,), jnp.float32),
    ],
    mesh=_MESH,
    scratch_types=[
        pltpu.VMEM((1, 128), jnp.int32),
        pltpu.VMEM((1, 128), jnp.int32),
        pltpu.VMEM((1, 64), jnp.int32),
        pltpu.VMEM((1, 64), jnp.int32),
        pltpu.VMEM((64, D), jnp.float32),
        pltpu.VMEM((64, D), jnp.float32),
        pltpu.VMEM((1, 128), jnp.float32),
        pltpu.VMEM((1, 64), jnp.float32),
        pltpu.VMEM((2000,), jnp.float32),
        pltpu.VMEM_SHARED((N0P,), jnp.float32),
        pltpu.VMEM_SHARED((N0P,), jnp.float32),
        pltpu.VMEM_SHARED((E0,), jnp.float32),
        pltpu.VMEM_SHARED((E0,), jnp.float32),
        pltpu.VMEM_SHARED((N0P, D), jnp.float32),
    ],
)


def _sc_agg0_body(m0, s0r, d0r,
                  agg0p,
                  sidx, didx, rows,
                  agg_sh):
    c = lax.axis_index("c")
    s = lax.axis_index("s")
    _zero_fill_2d(rows, 128, D)
    for j in range(5):
        pltpu.sync_copy(rows, agg_sh.at[pl.ds(s * 640 + 128 * j, 128)])
    plsc.subcore_barrier()

    def body(t, carry):
        r = s + 16 * t

        @pl.when(r < 625)
        def _():
            row = c * 625 + r
            pltpu.sync_copy(s0r.at[pl.ds(row, 1)], sidx)
            pltpu.sync_copy(d0r.at[pl.ds(row, 1)], didx)
            pltpu.sync_copy(m0.at[sidx.at[0]], rows)
            pltpu.sync_copy(rows, agg_sh.at[didx.at[0]], add=True)

        return carry

    lax.fori_loop(0, 40, body, 0)
    plsc.subcore_barrier()
    for j in range(5):
        pltpu.sync_copy(agg_sh.at[pl.ds(s * 640 + 128 * j, 128)], rows)
        pltpu.sync_copy(rows,
                        agg0p.at[pl.ds(c * N0P + s * 640 + 128 * j, 128)])


_sc_agg0 = pl.kernel(
    _sc_agg0_body,
    out_type=[jax.ShapeDtypeStruct((2 * N0P, D), jnp.float32)],
    mesh=_MESH,
    scratch_types=[
        pltpu.VMEM((1, 128), jnp.int32),
        pltpu.VMEM((1, 128), jnp.int32),
        pltpu.VMEM((128, D), jnp.float32),
        pltpu.VMEM_SHARED((N0P, D), jnp.float32),
    ],
)


def _sc_agg1_body(m1, s1r, d1r,
                  agg1,
                  sidx, didx, didx2, rows, zbuf,
                  acc_sh):
    c = lax.axis_index("c")
    s = lax.axis_index("s")
    iota = lax.iota(jnp.int32, 16)
    _zero_fill_2d(zbuf, 128, D)
    for p in range(8):
        b = c * 8 + p
        base = b * 10000
        for j in range(5):
            pltpu.sync_copy(zbuf, acc_sh.at[pl.ds(s * 640 + 128 * j, 128)])
        plsc.subcore_barrier()

        def body(t, carry):
            row = s + 16 * t
            pltpu.sync_copy(s1r.at[pl.ds(row, 1)], sidx)
            pltpu.sync_copy(d1r.at[pl.ds(row, 1)], didx)
            for k in range(8):
                dv = didx[0, pl.ds(16 * k, 16)]
                u = dv - base
                m = u.astype(jnp.uint32) < jnp.uint32(10000)
                didx2[0, pl.ds(16 * k, 16)] = jnp.where(m, u, 10200 + iota)
            pltpu.sync_copy(m1.at[sidx.at[0]], rows)
            pltpu.sync_copy(rows, acc_sh.at[didx2.at[0]], add=True)
            return carry

        lax.fori_loop(0, 625, body, 0)
        plsc.subcore_barrier()

        @pl.when(s < 15)
        def _():
            for j in range(5):
                pltpu.sync_copy(acc_sh.at[pl.ds(s * 640 + 128 * j, 128)], rows)
                pltpu.sync_copy(
                    rows, agg1.at[pl.ds(base + s * 640 + 128 * j, 128)])

        @pl.when(s == 15)
        def _():
            for (o, n) in ((0, 128), (128, 128), (256, 128)):
                pltpu.sync_copy(acc_sh.at[pl.ds(9600 + o, n)], rows)
                pltpu.sync_copy(rows, agg1.at[pl.ds(base + 9600 + o, n)])
            pltpu.sync_copy(acc_sh.at[pl.ds(9984, 16)], rows.at[pl.ds(0, 16)])
            pltpu.sync_copy(rows.at[pl.ds(0, 16)],
                            agg1.at[pl.ds(base + 9984, 16)])

        plsc.subcore_barrier()


_sc_agg1 = pl.kernel(
    _sc_agg1_body,
    out_type=[jax.ShapeDtypeStruct((E0, D), jnp.float32)],
    mesh=_MESH,
    scratch_types=[
        pltpu.VMEM((1, 128), jnp.int32),
        pltpu.VMEM((1, 128), jnp.int32),
        pltpu.VMEM((1, 128), jnp.int32),
        pltpu.VMEM((128, D), jnp.float32),
        pltpu.VMEM((128, D), jnp.float32),
        pltpu.VMEM_SHARED((N0P, D), jnp.float32),
    ],
)


def _tc_m0_body(h0_ref, g0p_ref, dout_ref, wc_ref, wf_ref, out_ref):
    deg = dout_ref[0] + dout_ref[1] + 1.0
    ns = lax.rsqrt(jnp.maximum(deg, 1.0))
    g = g0p_ref[0] + g0p_ref[1]
    z = (jnp.dot(h0_ref[...], wc_ref[...], preferred_element_type=jnp.float32)
         + jnp.dot(g, wf_ref[...], preferred_element_type=jnp.float32))
    out_ref[...] = z * ns


def _tc_m0(h0, g0p, dout0p, wc, wf):
    bn = 400
    grid = (N0 // bn,)
    return pl.pallas_call(
        _tc_m0_body,
        grid=grid,
        in_specs=[
            pl.BlockSpec((bn, D), lambda i: (i, 0)),
            pl.BlockSpec((2, bn, D), lambda i: (0, i, 0)),
            pl.BlockSpec((2, bn, 1), lambda i: (0, i, 0)),
            pl.BlockSpec((D, D), lambda i: (0, 0)),
            pl.BlockSpec((D, D), lambda i: (0, 0)),
        ],
        out_specs=pl.BlockSpec((bn, D), lambda i: (i, 0)),
        out_shape=jax.ShapeDtypeStruct((N0, D), jnp.float32),
    )(h0, g0p, dout0p, wc, wf)


def _tc_m1_body(h1_ref, gath_ref, dout_ref, wc_ref, wf_ref, out_ref):
    deg = dout_ref[0] + dout_ref[1] + 1.0
    ns = lax.rsqrt(jnp.maximum(deg, 1.0))
    z = (jnp.dot(h1_ref[...], wc_ref[...], preferred_element_type=jnp.float32)
         + jnp.dot(gath_ref[...], wf_ref[...], preferred_element_type=jnp.float32))
    out_ref[...] = z * ns


def _tc_m1(h1, gath, dout1p, wc, wf):
    bn = 640
    grid = (E0 // bn,)
    return pl.pallas_call(
        _tc_m1_body,
        grid=grid,
        in_specs=[
            pl.BlockSpec((bn, D), lambda i: (i, 0)),
            pl.BlockSpec((bn, D), lambda i: (i, 0)),
            pl.BlockSpec((2, bn, 1), lambda i: (0, i, 0)),
            pl.BlockSpec((D, D), lambda i: (0, 0)),
            pl.BlockSpec((D, D), lambda i: (0, 0)),
        ],
        out_specs=pl.BlockSpec((bn, D), lambda i: (i, 0)),
        out_shape=jax.ShapeDtypeStruct((E0, D), jnp.float32),
    )(h1, gath, dout1p, wc, wf)


def _ln_relu(x, nd, b, gamma, beta):
    y = nd * x + b
    mu = jnp.mean(y, axis=-1, keepdims=True)
    var = jnp.mean((y - mu) ** 2, axis=-1, keepdims=True)
    y = (y - mu) * lax.rsqrt(var + 1e-5) * gamma + beta
    return jnp.maximum(y, 0.0)


def _tc_r0_body(aggp_ref, m0_ref, din_ref, b_ref, gam_ref, bet_ref, out_ref):
    deg = din_ref[0] + din_ref[1] + 1.0
    nd = lax.rsqrt(jnp.maximum(deg, 1.0))
    x = aggp_ref[0] + aggp_ref[1] + m0_ref[...]
    out_ref[...] = _ln_relu(x, nd, b_ref[...], gam_ref[...], bet_ref[...])


def _tc_r0(agg0p, m0, din0p, b, gamma, beta):
    bn = 400
    grid = (N0 // bn,)
    return pl.pallas_call(
        _tc_r0_body,
        grid=grid,
        in_specs=[
            pl.BlockSpec((2, bn, D), lambda i: (0, i, 0)),
            pl.BlockSpec((bn, D), lambda i: (i, 0)),
            pl.BlockSpec((2, bn, 1), lambda i: (0, i, 0)),
            pl.BlockSpec((1, D), lambda i: (0, 0)),
            pl.BlockSpec((1, D), lambda i: (0, 0)),
            pl.BlockSpec((1, D), lambda i: (0, 0)),
        ],
        out_specs=pl.BlockSpec((bn, D), lambda i: (i, 0)),
        out_shape=jax.ShapeDtypeStruct((N0, D), jnp.float32),
    )(agg0p, m0, din0p, b, gamma, beta)


def _tc_r1_body(agg_ref, m1_ref, din_ref, b_ref, gam_ref, bet_ref, out_ref):
    deg = din_ref[0] + din_ref[1] + 1.0
    nd = lax.rsqrt(jnp.maximum(deg, 1.0))
    x = agg_ref[...] + m1_ref[...]
    out_ref[...] = _ln_relu(x, nd, b_ref[...], gam_ref[...], bet_ref[...])


def _tc_r1(agg1, m1, din1p, b, gamma, beta):
    bn = 640
    grid = (E0 // bn,)
    return pl.pallas_call(
        _tc_r1_body,
        grid=grid,
        in_specs=[
            pl.BlockSpec((bn, D), lambda i: (i, 0)),
            pl.BlockSpec((bn, D), lambda i: (i, 0)),
            pl.BlockSpec((2, bn, 1), lambda i: (0, i, 0)),
            pl.BlockSpec((1, D), lambda i: (0, 0)),
            pl.BlockSpec((1, D), lambda i: (0, 0)),
            pl.BlockSpec((1, D), lambda i: (0, 0)),
        ],
        out_specs=pl.BlockSpec((bn, D), lambda i: (i, 0)),
        out_shape=jax.ShapeDtypeStruct((E0, D), jnp.float32),
    )(agg1, m1, din1p, b, gamma, beta)


def kernel(h0, h1, edge_index0, edge_index1, params):
    src0, dst0 = edge_index0[0], edge_index0[1]
    src1, dst1 = edge_index1[0], edge_index1[1]
    s0r = src0.reshape(R0, 128)
    d0r = dst0.reshape(R0, 128)
    s0r64 = src0.reshape(2 * R0, 64)
    d0r64 = dst0.reshape(2 * R0, 64)
    s1r = src1.reshape(R1, 128)
    d1r = dst1.reshape(R1, 128)

    gath, g0p, dout0p, din0p, dout1p, din1p = _sc_pre(
        h0, h1, s1r, d1r, s0r64, d0r64)

    g0p = g0p.reshape(2, N0P, D)
    dout0p = dout0p.reshape(2, N0P, 1)
    din0p = din0p.reshape(2, N0P, 1)
    dout1p = dout1p.reshape(2, E0, 1)
    din1p = din1p.reshape(2, E0, 1)

    def fold(p):
        wc = p['Wc'] * p['conv_w'][None, :]
        wf = p['Wf'] * p['fuse_w'][None, :]
        b = (p['bc'] * p['conv_w'] + p['bf'] * p['fuse_w']).reshape(1, D)
        return wc, wf, b

    wc0, wf0, b0 = fold(params['td'])
    wc1, wf1, b1 = fold(params['bu'])

    m0 = _tc_m0(h0, g0p, dout0p[:, :N0], wc0, wf0)
    m1 = _tc_m1(h1, gath, dout1p, wc1, wf1)

    (agg0p,) = _sc_agg0(m0, s0r, d0r)
    (agg1,) = _sc_agg1(m1, s1r, d1r)

    r0 = _tc_r0(agg0p.reshape(2, N0P, D)[:, :N0], m0, din0p[:, :N0],
                b0, params['td']['gamma'].reshape(1, D),
                params['td']['beta'].reshape(1, D))
    r1 = _tc_r1(agg1, m1, din1p,
                b1, params['bu']['gamma'].reshape(1, D),
                params['bu']['beta'].reshape(1, D))
    return (r0, r1)


# speedup vs baseline: 1.5828x; 1.5828x over previous
import functools

import jax
import jax.numpy as jnp
from jax import lax
from jax.experimental import pallas as pl
from jax.experimental.pallas import tpu as pltpu
from jax.experimental.pallas import tpu_sc as plsc

N0 = 10000
E0 = 160000
E1 = 1280000
D = 128

N0P = 10240
R0 = E0 // 128
R1 = E1 // 128
NSLICE = 16
SW = D // NSLICE

_MESH = plsc.VectorSubcoreMesh(core_axis_name="c", subcore_axis_name="s")


def _fill_ones(ref):
    for k in range(8):
        ref[0, pl.ds(16 * k, 16)] = jnp.ones((16,), jnp.float32)


def _zero_fill_1d(ref, n):
    z = jnp.zeros((16,), jnp.float32)

    def b(i, carry):
        ref[pl.ds(16 * i, 16)] = z
        return carry

    lax.fori_loop(0, n // 16, b, 0)


def _zero_fill_2d(ref, rows, width):
    z = jnp.zeros((16,), jnp.float32)

    def b(r, carry):
        for k in range(width // 16):
            ref[r, pl.ds(16 * k, 16)] = z
        return carry

    lax.fori_loop(0, rows, b, 0)


def _sc_pre_body(h0, h1, s1r, d1r, s0r64, d0r64,
                 gath, g0p, dout0p, din0p, dout1p, din1p,
                 sidx, didx, sidx64, didx64, h1rows, h0rows, ones, ones64,
                 zbuf1d,
                 dout0_sh, din0_sh, dout1_sh, din1_sh, g0_sh):
    c = lax.axis_index("c")
    s = lax.axis_index("s")
    _fill_ones(ones)
    for k in range(4):
        ones64[0, pl.ds(16 * k, 16)] = jnp.ones((16,), jnp.float32)
    _zero_fill_1d(zbuf1d, 2000)
    _zero_fill_2d(h1rows, 64, D)
    pltpu.sync_copy(zbuf1d.at[pl.ds(0, 640)], dout0_sh.at[pl.ds(s * 640, 640)])
    pltpu.sync_copy(zbuf1d.at[pl.ds(0, 640)], din0_sh.at[pl.ds(s * 640, 640)])
    for j in range(5):
        pltpu.sync_copy(zbuf1d, dout1_sh.at[pl.ds(s * 10000 + 2000 * j, 2000)])
        pltpu.sync_copy(zbuf1d, din1_sh.at[pl.ds(s * 10000 + 2000 * j, 2000)])
    for j in range(10):
        pltpu.sync_copy(h1rows, g0_sh.at[pl.ds(s * 640 + 64 * j, 64)])
    plsc.subcore_barrier()

    def e1_body(t, carry):
        r = s + 16 * t

        @pl.when(r < 5000)
        def _():
            row = c * 5000 + r
            pltpu.sync_copy(s1r.at[pl.ds(row, 1)], sidx)
            pltpu.sync_copy(d1r.at[pl.ds(row, 1)], didx)
            pltpu.sync_copy(ones.at[0], dout1_sh.at[sidx.at[0]], add=True)
            pltpu.sync_copy(ones.at[0], din1_sh.at[didx.at[0]], add=True)

        return carry

    lax.fori_loop(0, 313, e1_body, 0)

    def e0_body(t, carry):
        r = s + 16 * t

        @pl.when(r < 1250)
        def _():
            row = c * 1250 + r
            pltpu.sync_copy(s0r64.at[pl.ds(row, 1)], sidx64)
            pltpu.sync_copy(d0r64.at[pl.ds(row, 1)], didx64)
            pltpu.sync_copy(ones64.at[0], dout0_sh.at[sidx64.at[0]], add=True)
            pltpu.sync_copy(ones64.at[0], din0_sh.at[didx64.at[0]], add=True)
            pltpu.sync_copy(h1.at[pl.ds(row * 64, 64)], h1rows)
            pltpu.sync_copy(h1rows, g0_sh.at[didx64.at[0]], add=True)
            pltpu.sync_copy(h0.at[didx64.at[0]], h0rows)
            pltpu.sync_copy(h0rows, gath.at[pl.ds(row * 64, 64)])

        return carry

    lax.fori_loop(0, 79, e0_body, 0)
    plsc.subcore_barrier()

    def _out1d_640(sh, out):
        pltpu.sync_copy(sh.at[pl.ds(s * 640, 640)], zbuf1d.at[pl.ds(0, 640)])
        pltpu.sync_copy(zbuf1d.at[pl.ds(0, 640)],
                        out.at[pl.ds(c * N0P + s * 640, 640)])

    def _out1d_10000(sh, out):
        for j in range(5):
            pltpu.sync_copy(sh.at[pl.ds(s * 10000 + 2000 * j, 2000)], zbuf1d)
            pltpu.sync_copy(
                zbuf1d, out.at[pl.ds(c * E0 + s * 10000 + 2000 * j, 2000)])

    _out1d_640(dout0_sh, dout0p)
    _out1d_640(din0_sh, din0p)
    _out1d_10000(dout1_sh, dout1p)
    _out1d_10000(din1_sh, din1p)
    for j in range(10):
        pltpu.sync_copy(g0_sh.at[pl.ds(s * 640 + 64 * j, 64)], h1rows)
        pltpu.sync_copy(h1rows,
                        g0p.at[pl.ds(c * N0P + s * 640 + 64 * j, 64)])


_sc_pre = pl.kernel(
    _sc_pre_body,
    out_type=[
        jax.ShapeDtypeStruct((E0, D), jnp.float32),
        jax.ShapeDtypeStruct((2 * N0P, D), jnp.float32),
        jax.ShapeDtypeStruct((2 * N0P,), jnp.float32),
        jax.ShapeDtypeStruct((2 * N0P,), jnp.float32),
        jax.ShapeDtypeStruct((2 * E0,), jnp.float32),
        jax.ShapeDtypeStruct((2 * E0,), jnp.float32),
    ],
    mesh=_MESH,
    scratch_types=[
        pltpu.VMEM((1, 128), jnp.int32),
        pltpu.VMEM((1, 128), jnp.int32),
        pltpu.VMEM((1, 64), jnp.int32),
        pltpu.VMEM((1, 64), jnp.int32),
        pltpu.VMEM((64, D), jnp.float32),
        pltpu.VMEM((64, D), jnp.float32),
        pltpu.VMEM((1, 128), jnp.float32),
        pltpu.VMEM((1, 64), jnp.float32),
        pltpu.VMEM((2000,), jnp.float32),
        pltpu.VMEM_SHARED((N0P,), jnp.float32),
        pltpu.VMEM_SHARED((N0P,), jnp.float32),
        pltpu.VMEM_SHARED((E0,), jnp.float32),
        pltpu.VMEM_SHARED((E0,), jnp.float32),
        pltpu.VMEM_SHARED((N0P, D), jnp.float32),
    ],
)


def _sc_agg0_body(m0, s0r, d0r,
                  agg0p,
                  sidx, didx, rows,
                  agg_sh):
    c = lax.axis_index("c")
    s = lax.axis_index("s")
    _zero_fill_2d(rows, 128, D)
    for j in range(5):
        pltpu.sync_copy(rows, agg_sh.at[pl.ds(s * 640 + 128 * j, 128)])
    plsc.subcore_barrier()

    def body(t, carry):
        r = s + 16 * t

        @pl.when(r < 625)
        def _():
            row = c * 625 + r
            pltpu.sync_copy(s0r.at[pl.ds(row, 1)], sidx)
            pltpu.sync_copy(d0r.at[pl.ds(row, 1)], didx)
            pltpu.sync_copy(m0.at[sidx.at[0]], rows)
            pltpu.sync_copy(rows, agg_sh.at[didx.at[0]], add=True)

        return carry

    lax.fori_loop(0, 40, body, 0)
    plsc.subcore_barrier()
    for j in range(5):
        pltpu.sync_copy(agg_sh.at[pl.ds(s * 640 + 128 * j, 128)], rows)
        pltpu.sync_copy(rows,
                        agg0p.at[pl.ds(c * N0P + s * 640 + 128 * j, 128)])


_sc_agg0 = pl.kernel(
    _sc_agg0_body,
    out_type=[jax.ShapeDtypeStruct((2 * N0P, D), jnp.float32)],
    mesh=_MESH,
    scratch_types=[
        pltpu.VMEM((1, 128), jnp.int32),
        pltpu.VMEM((1, 128), jnp.int32),
        pltpu.VMEM((128, D), jnp.float32),
        pltpu.VMEM_SHARED((N0P, D), jnp.float32),
    ],
)


def _sc_agg1_body(m1, s1r, d1r,
                  agg1,
                  sidxA, sidx2A, didxA, didx2A, rowsA,
                  sidxB, sidx2B, didxB, didx2B, rowsB,
                  semA, semB,
                  acc_sh):
    c = lax.axis_index("c")
    s = lax.axis_index("s")

    def transform(sidx, didx, sidx2, didx2, base):
        for k in range(8):
            dv = didx[0, pl.ds(16 * k, 16)]
            sv = sidx[0, pl.ds(16 * k, 16)]
            u = dv - base
            m = u.astype(jnp.uint32) < jnp.uint32(10000)
            didx2[0, pl.ds(16 * k, 16)] = jnp.where(m, u, dv & 8191)
            sidx2[0, pl.ds(16 * k, 16)] = jnp.where(m, sv, E0 + (dv & 1023))

    for p in range(8):
        b = c * 8 + p
        base = b * 10000
        _zero_fill_2d(rowsA, 128, D)
        for j in range(5):
            pltpu.sync_copy(rowsA, acc_sh.at[pl.ds(s * 640 + 128 * j, 128)])
        plsc.subcore_barrier()

        def body(i, carry):
            rowa = s + 16 * (2 * i)
            rowb = s + 16 * (2 * i + 1)
            pltpu.sync_copy(s1r.at[pl.ds(rowa, 1)], sidxA)
            pltpu.sync_copy(d1r.at[pl.ds(rowa, 1)], didxA)
            transform(sidxA, didxA, sidx2A, didx2A, base)
            pltpu.async_copy(m1.at[sidx2A.at[0]], rowsA, semA)

            @pl.when(i > 0)
            def _():
                pltpu.make_async_copy(m1.at[sidx2B.at[0]], rowsB, semB).wait()
                pltpu.sync_copy(rowsB, acc_sh.at[didx2B.at[0]], add=True)

            pltpu.sync_copy(s1r.at[pl.ds(rowb, 1)], sidxB)
            pltpu.sync_copy(d1r.at[pl.ds(rowb, 1)], didxB)
            transform(sidxB, didxB, sidx2B, didx2B, base)
            pltpu.async_copy(m1.at[sidx2B.at[0]], rowsB, semB)

            pltpu.make_async_copy(m1.at[sidx2A.at[0]], rowsA, semA).wait()
            pltpu.sync_copy(rowsA, acc_sh.at[didx2A.at[0]], add=True)
            return carry

        lax.fori_loop(0, 312, body, 0)
        pltpu.make_async_copy(m1.at[sidx2B.at[0]], rowsB, semB).wait()
        pltpu.sync_copy(rowsB, acc_sh.at[didx2B.at[0]], add=True)
        rowt = s + 16 * 624
        pltpu.sync_copy(s1r.at[pl.ds(rowt, 1)], sidxA)
        pltpu.sync_copy(d1r.at[pl.ds(rowt, 1)], didxA)
        transform(sidxA, didxA, sidx2A, didx2A, base)
        pltpu.sync_copy(m1.at[sidx2A.at[0]], rowsA)
        pltpu.sync_copy(rowsA, acc_sh.at[didx2A.at[0]], add=True)
        plsc.subcore_barrier()

        @pl.when(s < 15)
        def _():
            for j in range(5):
                pltpu.sync_copy(acc_sh.at[pl.ds(s * 640 + 128 * j, 128)], rowsA)
                pltpu.sync_copy(
                    rowsA, agg1.at[pl.ds(base + s * 640 + 128 * j, 128)])

        @pl.when(s == 15)
        def _():
            for (o, n) in ((0, 128), (128, 128), (256, 128)):
                pltpu.sync_copy(acc_sh.at[pl.ds(9600 + o, n)], rowsA)
                pltpu.sync_copy(rowsA, agg1.at[pl.ds(base + 9600 + o, n)])
            pltpu.sync_copy(acc_sh.at[pl.ds(9984, 16)], rowsA.at[pl.ds(0, 16)])
            pltpu.sync_copy(rowsA.at[pl.ds(0, 16)],
                            agg1.at[pl.ds(base + 9984, 16)])

        plsc.subcore_barrier()


_sc_agg1 = pl.kernel(
    _sc_agg1_body,
    out_type=[jax.ShapeDtypeStruct((E0, D), jnp.float32)],
    mesh=_MESH,
    scratch_types=[
        pltpu.VMEM((1, 128), jnp.int32),
        pltpu.VMEM((1, 128), jnp.int32),
        pltpu.VMEM((1, 128), jnp.int32),
        pltpu.VMEM((1, 128), jnp.int32),
        pltpu.VMEM((128, D), jnp.float32),
        pltpu.VMEM((1, 128), jnp.int32),
        pltpu.VMEM((1, 128), jnp.int32),
        pltpu.VMEM((1, 128), jnp.int32),
        pltpu.VMEM((1, 128), jnp.int32),
        pltpu.VMEM((128, D), jnp.float32),
        pltpu.SemaphoreType.DMA,
        pltpu.SemaphoreType.DMA,
        pltpu.VMEM_SHARED((N0P, D), jnp.float32),
    ],
)


def _tc_m0_body(h0_ref, g0p_ref, dout_ref, wc_ref, wf_ref, out_ref):
    deg = dout_ref[0] + dout_ref[1] + 1.0
    ns = lax.rsqrt(jnp.maximum(deg, 1.0))
    g = g0p_ref[0] + g0p_ref[1]
    z = (jnp.dot(h0_ref[...], wc_ref[...], preferred_element_type=jnp.float32)
         + jnp.dot(g, wf_ref[...], preferred_element_type=jnp.float32))
    out_ref[...] = z * ns


def _tc_m0(h0, g0p, dout0p, wc, wf):
    bn = 400
    grid = (N0 // bn,)
    return pl.pallas_call(
        _tc_m0_body,
        grid=grid,
        in_specs=[
            pl.BlockSpec((bn, D), lambda i: (i, 0)),
            pl.BlockSpec((2, bn, D), lambda i: (0, i, 0)),
            pl.BlockSpec((2, bn, 1), lambda i: (0, i, 0)),
            pl.BlockSpec((D, D), lambda i: (0, 0)),
            pl.BlockSpec((D, D), lambda i: (0, 0)),
        ],
        out_specs=pl.BlockSpec((bn, D), lambda i: (i, 0)),
        out_shape=jax.ShapeDtypeStruct((N0, D), jnp.float32),
    )(h0, g0p, dout0p, wc, wf)


def _tc_m1_body(h1_ref, gath_ref, dout_ref, wc_ref, wf_ref, out_ref):
    deg = dout_ref[0] + dout_ref[1] + 1.0
    ns = lax.rsqrt(jnp.maximum(deg, 1.0))
    z = (jnp.dot(h1_ref[...], wc_ref[...], preferred_element_type=jnp.float32)
         + jnp.dot(gath_ref[...], wf_ref[...], preferred_element_type=jnp.float32))
    out_ref[...] = z * ns


def _tc_m1(h1, gath, dout1p, wc, wf):
    bn = 640
    grid = (E0 // bn,)
    return pl.pallas_call(
        _tc_m1_body,
        grid=grid,
        in_specs=[
            pl.BlockSpec((bn, D), lambda i: (i, 0)),
            pl.BlockSpec((bn, D), lambda i: (i, 0)),
            pl.BlockSpec((2, bn, 1), lambda i: (0, i, 0)),
            pl.BlockSpec((D, D), lambda i: (0, 0)),
            pl.BlockSpec((D, D), lambda i: (0, 0)),
        ],
        out_specs=pl.BlockSpec((bn, D), lambda i: (i, 0)),
        out_shape=jax.ShapeDtypeStruct((E0, D), jnp.float32),
    )(h1, gath, dout1p, wc, wf)


def _ln_relu(x, nd, b, gamma, beta):
    y = nd * x + b
    mu = jnp.mean(y, axis=-1, keepdims=True)
    var = jnp.mean((y - mu) ** 2, axis=-1, keepdims=True)
    y = (y - mu) * lax.rsqrt(var + 1e-5) * gamma + beta
    return jnp.maximum(y, 0.0)


def _tc_r0_body(aggp_ref, m0_ref, din_ref, b_ref, gam_ref, bet_ref, out_ref):
    deg = din_ref[0] + din_ref[1] + 1.0
    nd = lax.rsqrt(jnp.maximum(deg, 1.0))
    x = aggp_ref[0] + aggp_ref[1] + m0_ref[...]
    out_ref[...] = _ln_relu(x, nd, b_ref[...], gam_ref[...], bet_ref[...])


def _tc_r0(agg0p, m0, din0p, b, gamma, beta):
    bn = 400
    grid = (N0 // bn,)
    return pl.pallas_call(
        _tc_r0_body,
        grid=grid,
        in_specs=[
            pl.BlockSpec((2, bn, D), lambda i: (0, i, 0)),
            pl.BlockSpec((bn, D), lambda i: (i, 0)),
            pl.BlockSpec((2, bn, 1), lambda i: (0, i, 0)),
            pl.BlockSpec((1, D), lambda i: (0, 0)),
            pl.BlockSpec((1, D), lambda i: (0, 0)),
            pl.BlockSpec((1, D), lambda i: (0, 0)),
        ],
        out_specs=pl.BlockSpec((bn, D), lambda i: (i, 0)),
        out_shape=jax.ShapeDtypeStruct((N0, D), jnp.float32),
    )(agg0p, m0, din0p, b, gamma, beta)


def _tc_r1_body(agg_ref, m1_ref, din_ref, b_ref, gam_ref, bet_ref, out_ref):
    deg = din_ref[0] + din_ref[1] + 1.0
    nd = lax.rsqrt(jnp.maximum(deg, 1.0))
    x = agg_ref[...] + m1_ref[...]
    out_ref[...] = _ln_relu(x, nd, b_ref[...], gam_ref[...], bet_ref[...])


def _tc_r1(agg1, m1, din1p, b, gamma, beta):
    bn = 640
    grid = (E0 // bn,)
    return pl.pallas_call(
        _tc_r1_body,
        grid=grid,
        in_specs=[
            pl.BlockSpec((bn, D), lambda i: (i, 0)),
            pl.BlockSpec((bn, D), lambda i: (i, 0)),
            pl.BlockSpec((2, bn, 1), lambda i: (0, i, 0)),
            pl.BlockSpec((1, D), lambda i: (0, 0)),
            pl.BlockSpec((1, D), lambda i: (0, 0)),
            pl.BlockSpec((1, D), lambda i: (0, 0)),
        ],
        out_specs=pl.BlockSpec((bn, D), lambda i: (i, 0)),
        out_shape=jax.ShapeDtypeStruct((E0, D), jnp.float32),
    )(agg1, m1, din1p, b, gamma, beta)


def kernel(h0, h1, edge_index0, edge_index1, params):
    src0, dst0 = edge_index0[0], edge_index0[1]
    src1, dst1 = edge_index1[0], edge_index1[1]
    s0r = src0.reshape(R0, 128)
    d0r = dst0.reshape(R0, 128)
    s0r64 = src0.reshape(2 * R0, 64)
    d0r64 = dst0.reshape(2 * R0, 64)
    s1r = src1.reshape(R1, 128)
    d1r = dst1.reshape(R1, 128)

    gath, g0p, dout0p, din0p, dout1p, din1p = _sc_pre(
        h0, h1, s1r, d1r, s0r64, d0r64)

    g0p = g0p.reshape(2, N0P, D)
    dout0p = dout0p.reshape(2, N0P, 1)
    din0p = din0p.reshape(2, N0P, 1)
    dout1p = dout1p.reshape(2, E0, 1)
    din1p = din1p.reshape(2, E0, 1)

    def fold(p):
        wc = p['Wc'] * p['conv_w'][None, :]
        wf = p['Wf'] * p['fuse_w'][None, :]
        b = (p['bc'] * p['conv_w'] + p['bf'] * p['fuse_w']).reshape(1, D)
        return wc, wf, b

    wc0, wf0, b0 = fold(params['td'])
    wc1, wf1, b1 = fold(params['bu'])

    m0 = _tc_m0(h0, g0p, dout0p[:, :N0], wc0, wf0)
    m1 = _tc_m1(h1, gath, dout1p, wc1, wf1)

    (agg0p,) = _sc_agg0(m0, s0r, d0r)
    m1p = jnp.concatenate([m1, jnp.zeros((1024, D), jnp.float32)], axis=0)
    (agg1,) = _sc_agg1(m1p, s1r, d1r)

    r0 = _tc_r0(agg0p.reshape(2, N0P, D)[:, :N0], m0, din0p[:, :N0],
                b0, params['td']['gamma'].reshape(1, D),
                params['td']['beta'].reshape(1, D))
    r1 = _tc_r1(agg1, m1, din1p,
                b1, params['bu']['gamma'].reshape(1, D),
                params['bu']['beta'].reshape(1, D))
    return (r0, r1)
